# Initial kernel scaffold; baseline (speedup 1.0000x reference)
import jax
import jax.numpy as jnp
from jax.experimental import pallas as pl


def kernel(points_xyz, features, indices, params):
    raise NotImplementedError("write your pallas kernel here")


# speedup vs baseline: 30.9112x; 30.9112x over previous
import functools

import jax
import jax.numpy as jnp
from jax import lax
from jax.experimental import pallas as pl
from jax.experimental.pallas import tpu as pltpu
from jax.experimental.pallas import tpu_sc as plsc

R0SQ = 0.2 * 0.2
R1SQ = 0.4 * 0.4
S0 = 32
S1 = 64
BN_EPS = 1e-5
NC = 2
NSUB = 16
NW = NC * NSUB
L = 16


def _pointfeat_body(x_ref, f_ref, w03_ref, wf0_ref, b0_ref, w13_ref, wf1_ref,
                    b1_ref, g0_ref, g1_ref):
    x = x_ref[...]
    f = f_ref[...]
    g0 = jnp.dot(x, w03_ref[...], preferred_element_type=jnp.float32)
    g0 = g0 + jnp.dot(f, wf0_ref[...], preferred_element_type=jnp.float32)
    g0_ref[:, :64] = g0 + b0_ref[...]
    g1 = jnp.dot(x, w13_ref[...], preferred_element_type=jnp.float32)
    g1 = g1 + jnp.dot(f, wf1_ref[...], preferred_element_type=jnp.float32)
    g1_ref[:, :96] = g1 + b1_ref[...]


def _d2_body(c_ref, p_ref, o_ref):
    c = c_ref[...]
    p = p_ref[...]
    cn = (c[:, 0] * c[:, 0] + c[:, 1] * c[:, 1]) + c[:, 2] * c[:, 2]
    pn = (p[:, 0] * p[:, 0] + p[:, 1] * p[:, 1]) + p[:, 2] * p[:, 2]
    mm = jnp.dot(c.astype(jnp.bfloat16), p.astype(jnp.bfloat16).T,
                 preferred_element_type=jnp.float32)
    o_ref[...] = (cn[:, None] + pn[None, :]) - 2.0 * mm


def _head_body(mb, nx_ref, r0_ref, r1_ref, w03_ref, w13_ref,
               w20_ref, b20_ref, w21_ref, b21_ref, o_ref):
    nx = nx_ref[...]
    t0 = jnp.dot(nx, w03_ref[...], preferred_element_type=jnp.float32)
    r0 = r0_ref[:, :64].reshape(mb, S0, 64)
    a0 = jnp.maximum(r0 - t0[:, None, :], 0.0).reshape(mb * S0, 64)
    h0 = jnp.dot(a0, w20_ref[...], preferred_element_type=jnp.float32)
    h0 = jnp.maximum(h0 + b20_ref[...], 0.0).reshape(mb, S0, 128)
    y0 = jnp.max(h0, axis=1)

    t1 = jnp.dot(nx, w13_ref[...], preferred_element_type=jnp.float32)
    r1 = r1_ref[:, :96].reshape(mb, S1, 96)
    a1 = jnp.maximum(r1 - t1[:, None, :], 0.0).reshape(mb * S1, 96)
    h1 = jnp.dot(a1, w21_ref[...], preferred_element_type=jnp.float32)
    h1 = jnp.maximum(h1 + b21_ref[...], 0.0).reshape(mb, S1, 128)
    y1 = jnp.max(h1, axis=1)

    y = jnp.concatenate([y0, y1], axis=1)
    o_ref[...] = y.T[None]


def _sc_centers_body(B, N, M, mpw, xyzt, cidx_hbm, nxyz_out,
                     x_v, y_v, z_v, cidx_v, nxb):
    wpb = NW // B
    wid = lax.axis_index("s") * NC + lax.axis_index("c")
    b = wid // wpb
    mbase = (wid % wpb) * mpw

    pltpu.sync_copy(xyzt.at[pl.ds(b * 3 * N, N)], x_v)
    pltpu.sync_copy(xyzt.at[pl.ds(b * 3 * N + N, N)], y_v)
    pltpu.sync_copy(xyzt.at[pl.ds(b * 3 * N + 2 * N, N)], z_v)
    pltpu.sync_copy(cidx_hbm.at[pl.ds(b * M + mbase, mpw)], cidx_v)

    iota = lax.broadcasted_iota(jnp.int32, (L,), 0)
    for g in range(mpw // L):
        iv = cidx_v[pl.ds(g * L, L)]
        gx = plsc.load_gather(x_v, [iv])
        gy = plsc.load_gather(y_v, [iv])
        gz = plsc.load_gather(z_v, [iv])
        pos = iota + g * L
        plsc.store_scatter(nxb, [pos, jnp.zeros((L,), jnp.int32)], gx)
        plsc.store_scatter(nxb, [pos, jnp.ones((L,), jnp.int32)], gy)
        plsc.store_scatter(nxb, [pos, jnp.full((L,), 2, jnp.int32)], gz)
    pltpu.sync_copy(nxb, nxyz_out.at[pl.ds(b * M + mbase, mpw)])


def _sc_select_body(B, N, M, mpw, d2_hbm, g0, g1, rows0_out, rows1_out,
                    d2_v, ib0, ib1, idx0_v, idx1_v, row0_v, row1_v, sem):
    wpb = NW // B
    wid = lax.axis_index("s") * NC + lax.axis_index("c")
    b = wid // wpb
    mbase = (wid % wpb) * mpw

    iota = lax.broadcasted_iota(jnp.int32, (L,), 0)
    nchunks = N // L
    nbase = b * N

    def center_body(i, carry):
        pltpu.sync_copy(d2_hbm.at[pl.ds((b * M + mbase + i) * N, N)], d2_v)

        def cond(c):
            j, w0, w1 = c
            return (j < nchunks) & ((w0 < S0) | (w1 < S1))

        def bodyw(c):
            j, w0, w1 = c
            d2 = d2_v[pl.ds(j * L, L)]
            nv = iota + (j * L)
            m0 = d2 < R0SQ
            m1 = d2 < R1SQ
            c0 = plsc.cumsum(m0.astype(jnp.int32))
            c1 = plsc.cumsum(m1.astype(jnp.int32))
            plsc.store_scatter(ib0, [jnp.minimum(w0 + c0 - 1, S0 + 31)],
                               nv, mask=m0)
            plsc.store_scatter(ib1, [jnp.minimum(w1 + c1 - 1, S1 + 31)],
                               nv, mask=m1)
            return j + 1, w0 + jnp.max(c0), w1 + jnp.max(c1)

        _, w0f, w1f = lax.while_loop(cond, bodyw, (0, 0, 0))

        cnt0 = jnp.minimum(w0f, S0)
        f0c = ib0[pl.ds(0, L)]
        first0 = jnp.sum(jnp.where(iota == 0, f0c, 0))
        for k in range(S0 // L):
            v = ib0[pl.ds(k * L, L)]
            posk = iota + k * L
            v = jnp.where(posk < cnt0, v, first0) + nbase
            idx0_v[pl.ds(i * S0 + k * L, L)] = v

        cnt1 = jnp.minimum(w1f, S1)
        f1c = ib1[pl.ds(0, L)]
        first1 = jnp.sum(jnp.where(iota == 0, f1c, 0))
        for k in range(S1 // L):
            v = ib1[pl.ds(k * L, L)]
            posk = iota + k * L
            v = jnp.where(posk < cnt1, v, first1) + nbase
            idx1_v[pl.ds(i * S1 + k * L, L)] = v
        return carry

    lax.fori_loop(0, mpw, center_body, 0)

    ch = 128
    rowbase0 = (b * M + mbase) * S0
    rowbase1 = (b * M + mbase) * S1

    def gat0(c, carry):
        pltpu.async_copy(g0.at[idx0_v.at[pl.ds(c * ch, ch)]], row0_v,
                         sem).wait()
        pltpu.sync_copy(row0_v, rows0_out.at[pl.ds(rowbase0 + c * ch, ch)])
        return carry

    lax.fori_loop(0, mpw * S0 // ch, gat0, 0)

    def gat1(c, carry):
        pltpu.async_copy(g1.at[idx1_v.at[pl.ds(c * ch, ch)]], row1_v,
                         sem).wait()
        pltpu.sync_copy(row1_v, rows1_out.at[pl.ds(rowbase1 + c * ch, ch)])
        return carry

    lax.fori_loop(0, mpw * S1 // ch, gat1, 0)


def _fold_bn(layer):
    s = layer['gamma'] * lax.rsqrt(layer['var'] + BN_EPS)
    return layer['W'] * s[:, None], (layer['b'] - layer['mean']) * s + layer['beta']


def kernel(points_xyz, features, indices, params):
    B, N, _ = points_xyz.shape
    M = indices.shape[1]
    C = features.shape[1]

    w10, b10 = _fold_bn(params[0][0])
    w20, b20 = _fold_bn(params[0][1])
    w11, b11 = _fold_bn(params[1][0])
    w21, b21 = _fold_bn(params[1][1])
    w03 = jnp.transpose(w10[:, :3])
    wf0 = jnp.transpose(w10[:, 3:])
    w13 = jnp.transpose(w11[:, :3])
    wf1 = jnp.transpose(w11[:, 3:])
    w2t0 = jnp.transpose(w20)
    w2t1 = jnp.transpose(w21)

    xyzr = points_xyz.reshape(B * N, 3)
    featr = jnp.transpose(features, (0, 2, 1)).reshape(B * N, C)
    xyzt = jnp.transpose(points_xyz, (0, 2, 1)).reshape(B * 3 * N)

    mpw = (B * M) // NW
    mesh = plsc.VectorSubcoreMesh(core_axis_name="c", subcore_axis_name="s")

    sc_centers = pl.kernel(
        functools.partial(_sc_centers_body, B, N, M, mpw),
        out_type=jax.ShapeDtypeStruct((B * M, 3), jnp.float32),
        mesh=mesh,
        compiler_params=pltpu.CompilerParams(needs_layout_passes=False),
        scratch_types=[
            pltpu.VMEM((N,), jnp.float32),
            pltpu.VMEM((N,), jnp.float32),
            pltpu.VMEM((N,), jnp.float32),
            pltpu.VMEM((mpw,), jnp.int32),
            pltpu.VMEM((mpw, 3), jnp.float32),
        ],
    )
    new_xyz_flat = sc_centers(xyzt, indices.reshape(B * M))

    nb = 2048
    g0, g1 = pl.pallas_call(
        _pointfeat_body,
        grid=(B * N // nb,),
        in_specs=[
            pl.BlockSpec((nb, 3), lambda i: (i, 0)),
            pl.BlockSpec((nb, C), lambda i: (i, 0)),
            pl.BlockSpec((3, 64), lambda i: (0, 0)),
            pl.BlockSpec((C, 64), lambda i: (0, 0)),
            pl.BlockSpec((1, 64), lambda i: (0, 0)),
            pl.BlockSpec((3, 96), lambda i: (0, 0)),
            pl.BlockSpec((C, 96), lambda i: (0, 0)),
            pl.BlockSpec((1, 96), lambda i: (0, 0)),
        ],
        out_specs=[
            pl.BlockSpec((nb, 128), lambda i: (i, 0)),
            pl.BlockSpec((nb, 128), lambda i: (i, 0)),
        ],
        out_shape=[
            jax.ShapeDtypeStruct((B * N, 128), jnp.float32),
            jax.ShapeDtypeStruct((B * N, 128), jnp.float32),
        ],
    )(xyzr, featr, w03, wf0, b10.reshape(1, 64), w13, wf1, b11.reshape(1, 96))

    mbd = 256
    mpb = M // mbd
    d2 = pl.pallas_call(
        _d2_body,
        grid=((B * M) // mbd,),
        in_specs=[
            pl.BlockSpec((mbd, 3), lambda i: (i, 0)),
            pl.BlockSpec((N, 3), lambda i, _q=mpb: (i // _q, 0)),
        ],
        out_specs=pl.BlockSpec((mbd, N), lambda i: (i, 0)),
        out_shape=jax.ShapeDtypeStruct((B * M, N), jnp.float32),
    )(new_xyz_flat, xyzr.reshape(B * N, 3))

    sc_select = pl.kernel(
        functools.partial(_sc_select_body, B, N, M, mpw),
        out_type=(
            jax.ShapeDtypeStruct((B * M * S0, 128), jnp.float32),
            jax.ShapeDtypeStruct((B * M * S1, 128), jnp.float32),
        ),
        mesh=mesh,
        compiler_params=pltpu.CompilerParams(needs_layout_passes=False),
        scratch_types=[
            pltpu.VMEM((N,), jnp.float32),
            pltpu.VMEM((S0 + 32,), jnp.int32),
            pltpu.VMEM((S1 + 32,), jnp.int32),
            pltpu.VMEM((mpw * S0,), jnp.int32),
            pltpu.VMEM((mpw * S1,), jnp.int32),
            pltpu.VMEM((128, 128), jnp.float32),
            pltpu.VMEM((128, 128), jnp.float32),
            pltpu.SemaphoreType.DMA,
        ],
    )
    rows0, rows1 = sc_select(d2.reshape(B * M * N), g0, g1)

    mb = 128
    m_blocks = (B * M) // mb
    out = pl.pallas_call(
        functools.partial(_head_body, mb),
        grid=(m_blocks,),
        in_specs=[
            pl.BlockSpec((mb, 3), lambda i: (i, 0)),
            pl.BlockSpec((mb * S0, 128), lambda i: (i, 0)),
            pl.BlockSpec((mb * S1, 128), lambda i: (i, 0)),
            pl.BlockSpec((3, 64), lambda i: (0, 0)),
            pl.BlockSpec((3, 96), lambda i: (0, 0)),
            pl.BlockSpec((64, 128), lambda i: (0, 0)),
            pl.BlockSpec((1, 128), lambda i: (0, 0)),
            pl.BlockSpec((96, 128), lambda i: (0, 0)),
            pl.BlockSpec((1, 128), lambda i: (0, 0)),
        ],
        out_specs=pl.BlockSpec(
            (1, 256, mb),
            lambda i, _mblk=M // mb: (i // _mblk, 0, i % _mblk)),
        out_shape=jax.ShapeDtypeStruct((B, 256, M), jnp.float32),
    )(new_xyz_flat, rows0, rows1, w03, w13,
      w2t0, b20.reshape(1, 128), w2t1, b21.reshape(1, 128))

    return (new_xyz_flat.reshape(B, M, 3), out, indices)


# speedup vs baseline: 41.1892x; 1.3325x over previous
import functools

import jax
import jax.numpy as jnp
from jax import lax
from jax.experimental import pallas as pl
from jax.experimental.pallas import tpu as pltpu
from jax.experimental.pallas import tpu_sc as plsc

R0SQ = 0.2 * 0.2
R1SQ = 0.4 * 0.4
S0 = 32
S1 = 64
BN_EPS = 1e-5
NC = 2
NSUB = 16
NW = NC * NSUB
L = 16


def _pointfeat_body(x_ref, f_ref, w03_ref, wf0_ref, b0_ref, w13_ref, wf1_ref,
                    b1_ref, g0_ref, g1_ref):
    x = x_ref[...]
    f = f_ref[...]
    g0 = jnp.dot(x, w03_ref[...], preferred_element_type=jnp.float32)
    g0 = g0 + jnp.dot(f, wf0_ref[...], preferred_element_type=jnp.float32)
    g0_ref[:, :64] = g0 + b0_ref[...]
    g1 = jnp.dot(x, w13_ref[...], preferred_element_type=jnp.float32)
    g1 = g1 + jnp.dot(f, wf1_ref[...], preferred_element_type=jnp.float32)
    g1_ref[:, :96] = g1 + b1_ref[...]


def _d2_body(c_ref, p_ref, o_ref):
    c = c_ref[...]
    p = p_ref[...]
    cn = (c[:, 0] * c[:, 0] + c[:, 1] * c[:, 1]) + c[:, 2] * c[:, 2]
    pn = (p[:, 0] * p[:, 0] + p[:, 1] * p[:, 1]) + p[:, 2] * p[:, 2]
    mm = jnp.dot(c.astype(jnp.bfloat16), p.astype(jnp.bfloat16).T,
                 preferred_element_type=jnp.float32)
    o_ref[...] = (cn[:, None] + pn[None, :]) - 2.0 * mm


def _head_body(mb, nx_ref, r0_ref, r1_ref, w03_ref, w13_ref,
               w20_ref, b20_ref, w21_ref, b21_ref, o_ref):
    nx = nx_ref[...]
    t0 = jnp.dot(nx, w03_ref[...], preferred_element_type=jnp.float32)
    r0 = r0_ref[:, :64].reshape(mb, S0, 64)
    a0 = jnp.maximum(r0 - t0[:, None, :], 0.0).reshape(mb * S0, 64)
    h0 = jnp.dot(a0, w20_ref[...], preferred_element_type=jnp.float32)
    h0 = jnp.maximum(h0 + b20_ref[...], 0.0).reshape(mb, S0, 128)
    y0 = jnp.max(h0, axis=1)

    t1 = jnp.dot(nx, w13_ref[...], preferred_element_type=jnp.float32)
    r1 = r1_ref[:, :96].reshape(mb, S1, 96)
    a1 = jnp.maximum(r1 - t1[:, None, :], 0.0).reshape(mb * S1, 96)
    h1 = jnp.dot(a1, w21_ref[...], preferred_element_type=jnp.float32)
    h1 = jnp.maximum(h1 + b21_ref[...], 0.0).reshape(mb, S1, 128)
    y1 = jnp.max(h1, axis=1)

    y = jnp.concatenate([y0, y1], axis=1)
    o_ref[...] = y.T[None]


def _sc_centers_body(B, N, M, mpw, xyzt, cidx_hbm, nxyz_out,
                     x_v, y_v, z_v, cidx_v, nxb):
    wpb = NW // B
    wid = lax.axis_index("s") * NC + lax.axis_index("c")
    b = wid // wpb
    mbase = (wid % wpb) * mpw

    pltpu.sync_copy(xyzt.at[pl.ds(b * 3 * N, N)], x_v)
    pltpu.sync_copy(xyzt.at[pl.ds(b * 3 * N + N, N)], y_v)
    pltpu.sync_copy(xyzt.at[pl.ds(b * 3 * N + 2 * N, N)], z_v)
    pltpu.sync_copy(cidx_hbm.at[pl.ds(b * M + mbase, mpw)], cidx_v)

    iota = lax.broadcasted_iota(jnp.int32, (L,), 0)
    for g in range(mpw // L):
        iv = cidx_v[pl.ds(g * L, L)]
        gx = plsc.load_gather(x_v, [iv])
        gy = plsc.load_gather(y_v, [iv])
        gz = plsc.load_gather(z_v, [iv])
        pos = iota + g * L
        plsc.store_scatter(nxb, [pos, jnp.zeros((L,), jnp.int32)], gx)
        plsc.store_scatter(nxb, [pos, jnp.ones((L,), jnp.int32)], gy)
        plsc.store_scatter(nxb, [pos, jnp.full((L,), 2, jnp.int32)], gz)
    pltpu.sync_copy(nxb, nxyz_out.at[pl.ds(b * M + mbase, mpw)])


def _sc_select_body(B, N, M, mpw, d2_hbm, g0, g1, rows0_out, rows1_out,
                    d2_a, d2_b, ib0, ib1, idx0_v, idx1_v,
                    row0_a, row0_b, row1_a, row1_b, sem_a, sem_b,
                    gsem_a, gsem_b):
    wpb = NW // B
    wid = lax.axis_index("s") * NC + lax.axis_index("c")
    b = wid // wpb
    mbase = (wid % wpb) * mpw
    rbase = b * M + mbase

    iota = lax.broadcasted_iota(jnp.int32, (L,), 0)
    nchunks = N // L
    nbase = b * N
    s0v = jnp.full((L,), S0, jnp.int32)
    s1v = jnp.full((L,), S1, jnp.int32)

    def scan_center(i, d2_v):
        def cond(c):
            j, w0, w1 = c
            return (j < nchunks) & jnp.any((w0 < s0v) | (w1 < s1v))

        def bodyw(c):
            j, w0, w1 = c
            d2 = d2_v[0, pl.ds(j * L, L)]
            nv = iota + (j * L)
            m0 = d2 < R0SQ
            m1 = d2 < R1SQ
            c0 = plsc.cumsum(m0.astype(jnp.int32))
            c1 = plsc.cumsum(m1.astype(jnp.int32))
            plsc.store_scatter(ib0, [jnp.minimum(w0 + c0 - 1, S0 + 31)],
                               nv, mask=m0)
            plsc.store_scatter(ib1, [jnp.minimum(w1 + c1 - 1, S1 + 31)],
                               nv, mask=m1)
            p0 = plsc.all_reduce_population_count(m0)
            p1 = plsc.all_reduce_population_count(m1)
            return j + 1, w0 + p0, w1 + p1

        zero = jnp.zeros((L,), jnp.int32)
        _, w0f, w1f = lax.while_loop(cond, bodyw, (0, zero, zero))

        cnt0 = jnp.minimum(w0f, s0v)
        f0c = ib0[pl.ds(0, L)]
        first0 = jnp.sum(jnp.where(iota == 0, f0c, 0))
        for k in range(S0 // L):
            v = ib0[pl.ds(k * L, L)]
            posk = iota + k * L
            v = jnp.where(posk < cnt0, v, first0) + nbase
            idx0_v[pl.ds(i * S0 + k * L, L)] = v

        cnt1 = jnp.minimum(w1f, s1v)
        f1c = ib1[pl.ds(0, L)]
        first1 = jnp.sum(jnp.where(iota == 0, f1c, 0))
        for k in range(S1 // L):
            v = ib1[pl.ds(k * L, L)]
            posk = iota + k * L
            v = jnp.where(posk < cnt1, v, first1) + nbase
            idx1_v[pl.ds(i * S1 + k * L, L)] = v

    pltpu.async_copy(d2_hbm.at[pl.ds(rbase, 1)], d2_a, sem_a)

    def pair_body(k, carry):
        i0 = 2 * k
        i1 = 2 * k + 1
        nxt = jnp.minimum(i1 + 1, mpw - 1)
        pltpu.async_copy(d2_hbm.at[pl.ds(rbase + i1, 1)], d2_b, sem_b)
        pltpu.make_async_copy(d2_hbm.at[pl.ds(rbase + i0, 1)], d2_a,
                              sem_a).wait()
        scan_center(i0, d2_a)
        pltpu.async_copy(d2_hbm.at[pl.ds(rbase + nxt, 1)], d2_a, sem_a)
        pltpu.make_async_copy(d2_hbm.at[pl.ds(rbase + i1, 1)], d2_b,
                              sem_b).wait()
        scan_center(i1, d2_b)
        return carry

    lax.fori_loop(0, mpw // 2, pair_body, 0)
    pltpu.make_async_copy(d2_hbm.at[pl.ds(rbase + mpw - 1, 1)], d2_a,
                          sem_a).wait()

    ch = 128

    def gat(idx_v, gtab, out, rowbase, rbuf_a, rbuf_b, nch):
        def gpair(k, carry):
            c0 = 2 * k
            c1 = 2 * k + 1
            ha = pltpu.async_copy(gtab.at[idx_v.at[pl.ds(c0 * ch, ch)]],
                                  rbuf_a, gsem_a)
            hb = pltpu.async_copy(gtab.at[idx_v.at[pl.ds(c1 * ch, ch)]],
                                  rbuf_b, gsem_b)
            ha.wait()
            pltpu.sync_copy(rbuf_a, out.at[pl.ds(rowbase + c0 * ch, ch)])
            hb.wait()
            pltpu.sync_copy(rbuf_b, out.at[pl.ds(rowbase + c1 * ch, ch)])
            return carry

        lax.fori_loop(0, nch // 2, gpair, 0)

    gat(idx0_v, g0, rows0_out, rbase * S0, row0_a, row0_b, mpw * S0 // ch)
    gat(idx1_v, g1, rows1_out, rbase * S1, row1_a, row1_b, mpw * S1 // ch)


def _fold_bn(layer):
    s = layer['gamma'] * lax.rsqrt(layer['var'] + BN_EPS)
    return layer['W'] * s[:, None], (layer['b'] - layer['mean']) * s + layer['beta']


def kernel(points_xyz, features, indices, params):
    B, N, _ = points_xyz.shape
    M = indices.shape[1]
    C = features.shape[1]

    w10, b10 = _fold_bn(params[0][0])
    w20, b20 = _fold_bn(params[0][1])
    w11, b11 = _fold_bn(params[1][0])
    w21, b21 = _fold_bn(params[1][1])
    w03 = jnp.transpose(w10[:, :3])
    wf0 = jnp.transpose(w10[:, 3:])
    w13 = jnp.transpose(w11[:, :3])
    wf1 = jnp.transpose(w11[:, 3:])
    w2t0 = jnp.transpose(w20)
    w2t1 = jnp.transpose(w21)

    xyzr = points_xyz.reshape(B * N, 3)
    featr = jnp.transpose(features, (0, 2, 1)).reshape(B * N, C)
    xyzt = jnp.transpose(points_xyz, (0, 2, 1)).reshape(B * 3 * N)

    mpw = (B * M) // NW
    mesh = plsc.VectorSubcoreMesh(core_axis_name="c", subcore_axis_name="s")

    sc_centers = pl.kernel(
        functools.partial(_sc_centers_body, B, N, M, mpw),
        out_type=jax.ShapeDtypeStruct((B * M, 3), jnp.float32),
        mesh=mesh,
        compiler_params=pltpu.CompilerParams(needs_layout_passes=False),
        scratch_types=[
            pltpu.VMEM((N,), jnp.float32),
            pltpu.VMEM((N,), jnp.float32),
            pltpu.VMEM((N,), jnp.float32),
            pltpu.VMEM((mpw,), jnp.int32),
            pltpu.VMEM((mpw, 3), jnp.float32),
        ],
    )
    new_xyz_flat = sc_centers(xyzt, indices.reshape(B * M))

    nb = 2048
    g0, g1 = pl.pallas_call(
        _pointfeat_body,
        grid=(B * N // nb,),
        in_specs=[
            pl.BlockSpec((nb, 3), lambda i: (i, 0)),
            pl.BlockSpec((nb, C), lambda i: (i, 0)),
            pl.BlockSpec((3, 64), lambda i: (0, 0)),
            pl.BlockSpec((C, 64), lambda i: (0, 0)),
            pl.BlockSpec((1, 64), lambda i: (0, 0)),
            pl.BlockSpec((3, 96), lambda i: (0, 0)),
            pl.BlockSpec((C, 96), lambda i: (0, 0)),
            pl.BlockSpec((1, 96), lambda i: (0, 0)),
        ],
        out_specs=[
            pl.BlockSpec((nb, 128), lambda i: (i, 0)),
            pl.BlockSpec((nb, 128), lambda i: (i, 0)),
        ],
        out_shape=[
            jax.ShapeDtypeStruct((B * N, 128), jnp.float32),
            jax.ShapeDtypeStruct((B * N, 128), jnp.float32),
        ],
    )(xyzr, featr, w03, wf0, b10.reshape(1, 64), w13, wf1, b11.reshape(1, 96))

    mbd = 256
    mpb = M // mbd
    d2 = pl.pallas_call(
        _d2_body,
        grid=((B * M) // mbd,),
        in_specs=[
            pl.BlockSpec((mbd, 3), lambda i: (i, 0)),
            pl.BlockSpec((N, 3), lambda i, _q=mpb: (i // _q, 0)),
        ],
        out_specs=pl.BlockSpec((mbd, N), lambda i: (i, 0)),
        out_shape=jax.ShapeDtypeStruct((B * M, N), jnp.float32),
    )(new_xyz_flat, xyzr.reshape(B * N, 3))

    sc_select = pl.kernel(
        functools.partial(_sc_select_body, B, N, M, mpw),
        out_type=(
            jax.ShapeDtypeStruct((B * M * S0, 128), jnp.float32),
            jax.ShapeDtypeStruct((B * M * S1, 128), jnp.float32),
        ),
        mesh=mesh,
        compiler_params=pltpu.CompilerParams(needs_layout_passes=False),
        scratch_types=[
            pltpu.VMEM((1, N), jnp.float32),
            pltpu.VMEM((1, N), jnp.float32),
            pltpu.VMEM((S0 + 32,), jnp.int32),
            pltpu.VMEM((S1 + 32,), jnp.int32),
            pltpu.VMEM((mpw * S0,), jnp.int32),
            pltpu.VMEM((mpw * S1,), jnp.int32),
            pltpu.VMEM((128, 128), jnp.float32),
            pltpu.VMEM((128, 128), jnp.float32),
            pltpu.VMEM((128, 128), jnp.float32),
            pltpu.VMEM((128, 128), jnp.float32),
            pltpu.SemaphoreType.DMA,
            pltpu.SemaphoreType.DMA,
            pltpu.SemaphoreType.DMA,
            pltpu.SemaphoreType.DMA,
        ],
    )
    rows0, rows1 = sc_select(d2, g0, g1)

    mb = 128
    m_blocks = (B * M) // mb
    out = pl.pallas_call(
        functools.partial(_head_body, mb),
        grid=(m_blocks,),
        in_specs=[
            pl.BlockSpec((mb, 3), lambda i: (i, 0)),
            pl.BlockSpec((mb * S0, 128), lambda i: (i, 0)),
            pl.BlockSpec((mb * S1, 128), lambda i: (i, 0)),
            pl.BlockSpec((3, 64), lambda i: (0, 0)),
            pl.BlockSpec((3, 96), lambda i: (0, 0)),
            pl.BlockSpec((64, 128), lambda i: (0, 0)),
            pl.BlockSpec((1, 128), lambda i: (0, 0)),
            pl.BlockSpec((96, 128), lambda i: (0, 0)),
            pl.BlockSpec((1, 128), lambda i: (0, 0)),
        ],
        out_specs=pl.BlockSpec(
            (1, 256, mb),
            lambda i, _mblk=M // mb: (i // _mblk, 0, i % _mblk)),
        out_shape=jax.ShapeDtypeStruct((B, 256, M), jnp.float32),
    )(new_xyz_flat, rows0, rows1, w03, w13,
      w2t0, b20.reshape(1, 128), w2t1, b21.reshape(1, 128))

    return (new_xyz_flat.reshape(B, M, 3), out, indices)


# speedup vs baseline: 41.5136x; 1.0079x over previous
import functools

import jax
import jax.numpy as jnp
from jax import lax
from jax.experimental import pallas as pl
from jax.experimental.pallas import tpu as pltpu
from jax.experimental.pallas import tpu_sc as plsc

R0SQ = 0.2 * 0.2
R1SQ = 0.4 * 0.4
S0 = 32
S1 = 64
BN_EPS = 1e-5
NC = 2
NSUB = 16
NW = NC * NSUB
L = 16


def _pointfeat_body(x_ref, f_ref, w03_ref, wf0_ref, b0_ref, w13_ref, wf1_ref,
                    b1_ref, g0_ref, g1_ref):
    x = x_ref[...]
    f = f_ref[...]
    g0 = jnp.dot(x, w03_ref[...], preferred_element_type=jnp.float32)
    g0 = g0 + jnp.dot(f, wf0_ref[...], preferred_element_type=jnp.float32)
    g0_ref[:, :64] = g0 + b0_ref[...]
    g1 = jnp.dot(x, w13_ref[...], preferred_element_type=jnp.float32)
    g1 = g1 + jnp.dot(f, wf1_ref[...], preferred_element_type=jnp.float32)
    g1_ref[:, :96] = g1 + b1_ref[...]


def _head_body(mb, nx_ref, r0_ref, r1_ref, w03_ref, w13_ref,
               w20_ref, b20_ref, w21_ref, b21_ref, o_ref):
    nx = nx_ref[...]
    t0 = jnp.dot(nx, w03_ref[...], preferred_element_type=jnp.float32)
    r0 = r0_ref[:, :64].reshape(mb, S0, 64)
    a0 = jnp.maximum(r0 - t0[:, None, :], 0.0).reshape(mb * S0, 64)
    h0 = jnp.dot(a0, w20_ref[...], preferred_element_type=jnp.float32)
    h0 = jnp.maximum(h0 + b20_ref[...], 0.0).reshape(mb, S0, 128)
    y0 = jnp.max(h0, axis=1)

    t1 = jnp.dot(nx, w13_ref[...], preferred_element_type=jnp.float32)
    r1 = r1_ref[:, :96].reshape(mb, S1, 96)
    a1 = jnp.maximum(r1 - t1[:, None, :], 0.0).reshape(mb * S1, 96)
    h1 = jnp.dot(a1, w21_ref[...], preferred_element_type=jnp.float32)
    h1 = jnp.maximum(h1 + b21_ref[...], 0.0).reshape(mb, S1, 128)
    y1 = jnp.max(h1, axis=1)

    y = jnp.concatenate([y0, y1], axis=1)
    o_ref[...] = y.T[None]


def _round_bf16(x):
    u = plsc.bitcast(x, jnp.int32)
    r = (u + 0x7FFF + (lax.shift_right_logical(u, 16) & 1)) & (-65536)
    return plsc.bitcast(r, jnp.float32)


def _sc_centers_body(B, N, M, mpw, xyzt, cidx_hbm, nxyz_out, aux_out,
                     x_v, y_v, z_v, cidx_v, nxb, cxb_v, cyb_v, czb_v, cn_v):
    wpb = NW // B
    wid = lax.axis_index("s") * NC + lax.axis_index("c")
    b = wid // wpb
    mbase = (wid % wpb) * mpw

    pltpu.sync_copy(xyzt.at[pl.ds(b * 3 * N, N)], x_v)
    pltpu.sync_copy(xyzt.at[pl.ds(b * 3 * N + N, N)], y_v)
    pltpu.sync_copy(xyzt.at[pl.ds(b * 3 * N + 2 * N, N)], z_v)
    pltpu.sync_copy(cidx_hbm.at[pl.ds(b * M + mbase, mpw)], cidx_v)

    iota = lax.broadcasted_iota(jnp.int32, (L,), 0)
    for g in range(mpw // L):
        iv = cidx_v[pl.ds(g * L, L)]
        gx = plsc.load_gather(x_v, [iv])
        gy = plsc.load_gather(y_v, [iv])
        gz = plsc.load_gather(z_v, [iv])
        pos = iota + g * L
        plsc.store_scatter(nxb, [pos, jnp.zeros((L,), jnp.int32)], gx)
        plsc.store_scatter(nxb, [pos, jnp.ones((L,), jnp.int32)], gy)
        plsc.store_scatter(nxb, [pos, jnp.full((L,), 2, jnp.int32)], gz)
        sl = pl.ds(g * L, L)
        cxb_v[sl] = _round_bf16(gx)
        cyb_v[sl] = _round_bf16(gy)
        czb_v[sl] = _round_bf16(gz)
        cn_v[sl] = (gx * gx + gy * gy) + gz * gz
    pltpu.sync_copy(nxb, nxyz_out.at[pl.ds(b * M + mbase, mpw)])
    bm = B * M
    pltpu.sync_copy(cxb_v, aux_out.at[pl.ds(b * M + mbase, mpw)])
    pltpu.sync_copy(cyb_v, aux_out.at[pl.ds(bm + b * M + mbase, mpw)])
    pltpu.sync_copy(czb_v, aux_out.at[pl.ds(2 * bm + b * M + mbase, mpw)])
    pltpu.sync_copy(cn_v, aux_out.at[pl.ds(3 * bm + b * M + mbase, mpw)])


def _sc_select_body(B, N, M, mpw, xyzt, aux_hbm, g0, g1, rows0_out, rows1_out,
                    xb_v, yb_v, zb_v, pn_v, cxb_c, cyb_c, czb_c, cn_c,
                    ib0, ib1, idx0_v, idx1_v,
                    row0_a, row0_b, row1_a, row1_b,
                    gsem_a, gsem_b):
    wpb = NW // B
    wid = lax.axis_index("s") * NC + lax.axis_index("c")
    b = wid // wpb
    mbase = (wid % wpb) * mpw
    rbase = b * M + mbase
    bm = B * M

    pltpu.sync_copy(xyzt.at[pl.ds(b * 3 * N, N)], xb_v)
    pltpu.sync_copy(xyzt.at[pl.ds(b * 3 * N + N, N)], yb_v)
    pltpu.sync_copy(xyzt.at[pl.ds(b * 3 * N + 2 * N, N)], zb_v)
    pltpu.sync_copy(aux_hbm.at[pl.ds(rbase, mpw)], cxb_c)
    pltpu.sync_copy(aux_hbm.at[pl.ds(bm + rbase, mpw)], cyb_c)
    pltpu.sync_copy(aux_hbm.at[pl.ds(2 * bm + rbase, mpw)], czb_c)
    pltpu.sync_copy(aux_hbm.at[pl.ds(3 * bm + rbase, mpw)], cn_c)

    iota = lax.broadcasted_iota(jnp.int32, (L,), 0)
    nchunks = N // L
    nbase = b * N
    s0v = jnp.full((L,), S0, jnp.int32)
    s1v = jnp.full((L,), S1, jnp.int32)

    def prep(j, carry):
        sl = pl.ds(j * L, L)
        xs = xb_v[sl]
        ys = yb_v[sl]
        zs = zb_v[sl]
        pn_v[sl] = (xs * xs + ys * ys) + zs * zs
        xb_v[sl] = _round_bf16(xs)
        yb_v[sl] = _round_bf16(ys)
        zb_v[sl] = _round_bf16(zs)
        return carry

    lax.fori_loop(0, nchunks, prep, 0)

    def scan_center(i, cx, cy, cz, cn):
        def cond(c):
            j, w0, w1 = c
            return (j < nchunks) & jnp.any((w0 < s0v) | (w1 < s1v))

        def bodyw(c):
            j, w0, w1 = c
            sl = pl.ds(j * L, L)
            mm = (cx * xb_v[sl] + cy * yb_v[sl]) + cz * zb_v[sl]
            d2 = (cn + pn_v[sl]) - 2.0 * mm
            nv = iota + (j * L)
            m0 = d2 < R0SQ
            m1 = d2 < R1SQ
            c0 = plsc.cumsum(m0.astype(jnp.int32))
            c1 = plsc.cumsum(m1.astype(jnp.int32))
            plsc.store_scatter(ib0, [jnp.minimum(w0 + c0 - 1, S0 + 31)],
                               nv, mask=m0)
            plsc.store_scatter(ib1, [jnp.minimum(w1 + c1 - 1, S1 + 31)],
                               nv, mask=m1)
            p0 = plsc.all_reduce_population_count(m0)
            p1 = plsc.all_reduce_population_count(m1)
            return j + 1, w0 + p0, w1 + p1

        zero = jnp.zeros((L,), jnp.int32)
        _, w0f, w1f = lax.while_loop(cond, bodyw, (0, zero, zero))

        cnt0 = jnp.minimum(w0f, s0v)
        f0c = ib0[pl.ds(0, L)]
        first0 = jnp.sum(jnp.where(iota == 0, f0c, 0))
        for k in range(S0 // L):
            v = ib0[pl.ds(k * L, L)]
            posk = iota + k * L
            v = jnp.where(posk < cnt0, v, first0) + nbase
            idx0_v[pl.ds(i * S0 + k * L, L)] = v

        cnt1 = jnp.minimum(w1f, s1v)
        f1c = ib1[pl.ds(0, L)]
        first1 = jnp.sum(jnp.where(iota == 0, f1c, 0))
        for k in range(S1 // L):
            v = ib1[pl.ds(k * L, L)]
            posk = iota + k * L
            v = jnp.where(posk < cnt1, v, first1) + nbase
            idx1_v[pl.ds(i * S1 + k * L, L)] = v

    def center_body(i, carry):
        g = i // L
        lane = i % L
        sel = iota == lane
        gs = pl.ds(g * L, L)
        cx = jnp.sum(jnp.where(sel, cxb_c[gs], 0.0))
        cy = jnp.sum(jnp.where(sel, cyb_c[gs], 0.0))
        cz = jnp.sum(jnp.where(sel, czb_c[gs], 0.0))
        cn = jnp.sum(jnp.where(sel, cn_c[gs], 0.0))
        scan_center(i, cx, cy, cz, cn)
        return carry

    lax.fori_loop(0, mpw, center_body, 0)

    ch = 128

    def gat(idx_v, gtab, out, rowbase, rbuf_a, rbuf_b, nch):
        def gpair(k, carry):
            c0 = 2 * k
            c1 = 2 * k + 1
            ha = pltpu.async_copy(gtab.at[idx_v.at[pl.ds(c0 * ch, ch)]],
                                  rbuf_a, gsem_a)
            hb = pltpu.async_copy(gtab.at[idx_v.at[pl.ds(c1 * ch, ch)]],
                                  rbuf_b, gsem_b)
            ha.wait()
            pltpu.sync_copy(rbuf_a, out.at[pl.ds(rowbase + c0 * ch, ch)])
            hb.wait()
            pltpu.sync_copy(rbuf_b, out.at[pl.ds(rowbase + c1 * ch, ch)])
            return carry

        lax.fori_loop(0, nch // 2, gpair, 0)

    gat(idx0_v, g0, rows0_out, rbase * S0, row0_a, row0_b, mpw * S0 // ch)
    gat(idx1_v, g1, rows1_out, rbase * S1, row1_a, row1_b, mpw * S1 // ch)


def _fold_bn(layer):
    s = layer['gamma'] * lax.rsqrt(layer['var'] + BN_EPS)
    return layer['W'] * s[:, None], (layer['b'] - layer['mean']) * s + layer['beta']


def kernel(points_xyz, features, indices, params):
    B, N, _ = points_xyz.shape
    M = indices.shape[1]
    C = features.shape[1]

    w10, b10 = _fold_bn(params[0][0])
    w20, b20 = _fold_bn(params[0][1])
    w11, b11 = _fold_bn(params[1][0])
    w21, b21 = _fold_bn(params[1][1])
    w03 = jnp.transpose(w10[:, :3])
    wf0 = jnp.transpose(w10[:, 3:])
    w13 = jnp.transpose(w11[:, :3])
    wf1 = jnp.transpose(w11[:, 3:])
    w2t0 = jnp.transpose(w20)
    w2t1 = jnp.transpose(w21)

    xyzr = points_xyz.reshape(B * N, 3)
    featr = jnp.transpose(features, (0, 2, 1)).reshape(B * N, C)
    xyzt = jnp.transpose(points_xyz, (0, 2, 1)).reshape(B * 3 * N)

    mpw = (B * M) // NW
    mesh = plsc.VectorSubcoreMesh(core_axis_name="c", subcore_axis_name="s")

    sc_centers = pl.kernel(
        functools.partial(_sc_centers_body, B, N, M, mpw),
        out_type=(
            jax.ShapeDtypeStruct((B * M, 3), jnp.float32),
            jax.ShapeDtypeStruct((4 * B * M,), jnp.float32),
        ),
        mesh=mesh,
        compiler_params=pltpu.CompilerParams(needs_layout_passes=False),
        scratch_types=[
            pltpu.VMEM((N,), jnp.float32),
            pltpu.VMEM((N,), jnp.float32),
            pltpu.VMEM((N,), jnp.float32),
            pltpu.VMEM((mpw,), jnp.int32),
            pltpu.VMEM((mpw, 3), jnp.float32),
            pltpu.VMEM((mpw,), jnp.float32),
            pltpu.VMEM((mpw,), jnp.float32),
            pltpu.VMEM((mpw,), jnp.float32),
            pltpu.VMEM((mpw,), jnp.float32),
        ],
    )
    new_xyz_flat, aux = sc_centers(xyzt, indices.reshape(B * M))

    nb = 2048
    g0, g1 = pl.pallas_call(
        _pointfeat_body,
        grid=(B * N // nb,),
        in_specs=[
            pl.BlockSpec((nb, 3), lambda i: (i, 0)),
            pl.BlockSpec((nb, C), lambda i: (i, 0)),
            pl.BlockSpec((3, 64), lambda i: (0, 0)),
            pl.BlockSpec((C, 64), lambda i: (0, 0)),
            pl.BlockSpec((1, 64), lambda i: (0, 0)),
            pl.BlockSpec((3, 96), lambda i: (0, 0)),
            pl.BlockSpec((C, 96), lambda i: (0, 0)),
            pl.BlockSpec((1, 96), lambda i: (0, 0)),
        ],
        out_specs=[
            pl.BlockSpec((nb, 128), lambda i: (i, 0)),
            pl.BlockSpec((nb, 128), lambda i: (i, 0)),
        ],
        out_shape=[
            jax.ShapeDtypeStruct((B * N, 128), jnp.float32),
            jax.ShapeDtypeStruct((B * N, 128), jnp.float32),
        ],
    )(xyzr, featr, w03, wf0, b10.reshape(1, 64), w13, wf1, b11.reshape(1, 96))

    sc_select = pl.kernel(
        functools.partial(_sc_select_body, B, N, M, mpw),
        out_type=(
            jax.ShapeDtypeStruct((B * M * S0, 128), jnp.float32),
            jax.ShapeDtypeStruct((B * M * S1, 128), jnp.float32),
        ),
        mesh=mesh,
        compiler_params=pltpu.CompilerParams(needs_layout_passes=False),
        scratch_types=[
            pltpu.VMEM((N,), jnp.float32),
            pltpu.VMEM((N,), jnp.float32),
            pltpu.VMEM((N,), jnp.float32),
            pltpu.VMEM((N,), jnp.float32),
            pltpu.VMEM((mpw,), jnp.float32),
            pltpu.VMEM((mpw,), jnp.float32),
            pltpu.VMEM((mpw,), jnp.float32),
            pltpu.VMEM((mpw,), jnp.float32),
            pltpu.VMEM((S0 + 32,), jnp.int32),
            pltpu.VMEM((S1 + 32,), jnp.int32),
            pltpu.VMEM((mpw * S0,), jnp.int32),
            pltpu.VMEM((mpw * S1,), jnp.int32),
            pltpu.VMEM((128, 128), jnp.float32),
            pltpu.VMEM((128, 128), jnp.float32),
            pltpu.VMEM((128, 128), jnp.float32),
            pltpu.VMEM((128, 128), jnp.float32),
            pltpu.SemaphoreType.DMA,
            pltpu.SemaphoreType.DMA,
        ],
    )
    rows0, rows1 = sc_select(xyzt, aux, g0, g1)

    mb = 128
    m_blocks = (B * M) // mb
    out = pl.pallas_call(
        functools.partial(_head_body, mb),
        grid=(m_blocks,),
        in_specs=[
            pl.BlockSpec((mb, 3), lambda i: (i, 0)),
            pl.BlockSpec((mb * S0, 128), lambda i: (i, 0)),
            pl.BlockSpec((mb * S1, 128), lambda i: (i, 0)),
            pl.BlockSpec((3, 64), lambda i: (0, 0)),
            pl.BlockSpec((3, 96), lambda i: (0, 0)),
            pl.BlockSpec((64, 128), lambda i: (0, 0)),
            pl.BlockSpec((1, 128), lambda i: (0, 0)),
            pl.BlockSpec((96, 128), lambda i: (0, 0)),
            pl.BlockSpec((1, 128), lambda i: (0, 0)),
        ],
        out_specs=pl.BlockSpec(
            (1, 256, mb),
            lambda i, _mblk=M // mb: (i // _mblk, 0, i % _mblk)),
        out_shape=jax.ShapeDtypeStruct((B, 256, M), jnp.float32),
    )(new_xyz_flat, rows0, rows1, w03, w13,
      w2t0, b20.reshape(1, 128), w2t1, b21.reshape(1, 128))

    return (new_xyz_flat.reshape(B, M, 3), out, indices)


# speedup vs baseline: 48.3234x; 1.1640x over previous
import functools

import jax
import jax.numpy as jnp
from jax import lax
from jax.experimental import pallas as pl
from jax.experimental.pallas import tpu as pltpu
from jax.experimental.pallas import tpu_sc as plsc

R0SQ = 0.2 * 0.2
R1SQ = 0.4 * 0.4
S0 = 32
S1 = 64
BN_EPS = 1e-5
NC = 2
NSUB = 16
NW = NC * NSUB
L = 16


def _pointfeat_body(x_ref, f_ref, w03_ref, wf0_ref, b0_ref, w13_ref, wf1_ref,
                    b1_ref, g0_ref, g1_ref):
    x = x_ref[...]
    f = f_ref[...]
    g0 = jnp.dot(x, w03_ref[...], preferred_element_type=jnp.float32)
    g0 = g0 + jnp.dot(f, wf0_ref[...], preferred_element_type=jnp.float32)
    g0_ref[:, :64] = g0 + b0_ref[...]
    g1 = jnp.dot(x, w13_ref[...], preferred_element_type=jnp.float32)
    g1 = g1 + jnp.dot(f, wf1_ref[...], preferred_element_type=jnp.float32)
    g1_ref[:, :96] = g1 + b1_ref[...]


def _head_body(mb, nx_ref, r0_ref, r1_ref, w03_ref, w13_ref,
               w20_ref, b20_ref, w21_ref, b21_ref, o_ref):
    nx = nx_ref[...]
    t0 = jnp.dot(nx, w03_ref[...], preferred_element_type=jnp.float32)
    r0 = r0_ref[:, :64].reshape(mb, S0, 64)
    a0 = jnp.maximum(r0 - t0[:, None, :], 0.0).reshape(mb * S0, 64)
    h0 = jnp.dot(a0, w20_ref[...], preferred_element_type=jnp.float32)
    h0 = jnp.maximum(h0 + b20_ref[...], 0.0).reshape(mb, S0, 128)
    y0 = jnp.max(h0, axis=1)

    t1 = jnp.dot(nx, w13_ref[...], preferred_element_type=jnp.float32)
    r1 = r1_ref[:, :96].reshape(mb, S1, 96)
    a1 = jnp.maximum(r1 - t1[:, None, :], 0.0).reshape(mb * S1, 96)
    h1 = jnp.dot(a1, w21_ref[...], preferred_element_type=jnp.float32)
    h1 = jnp.maximum(h1 + b21_ref[...], 0.0).reshape(mb, S1, 128)
    y1 = jnp.max(h1, axis=1)

    y = jnp.concatenate([y0, y1], axis=1)
    o_ref[...] = y.T[None]


def _round_bf16(x):
    u = plsc.bitcast(x, jnp.int32)
    r = (u + 0x7FFF + (lax.shift_right_logical(u, 16) & 1)) & (-65536)
    return plsc.bitcast(r, jnp.float32)


def _sc_centers_body(B, N, M, mpw, xyzt, cidx_hbm, nxyz_out, aux_out,
                     x_v, y_v, z_v, cidx_v, nxb, cxb_v, cyb_v, czb_v, cn_v):
    wpb = NW // B
    wid = lax.axis_index("s") * NC + lax.axis_index("c")
    b = wid // wpb
    mbase = (wid % wpb) * mpw

    pltpu.sync_copy(xyzt.at[pl.ds(b * 3 * N, N)], x_v)
    pltpu.sync_copy(xyzt.at[pl.ds(b * 3 * N + N, N)], y_v)
    pltpu.sync_copy(xyzt.at[pl.ds(b * 3 * N + 2 * N, N)], z_v)
    pltpu.sync_copy(cidx_hbm.at[pl.ds(b * M + mbase, mpw)], cidx_v)

    iota = lax.broadcasted_iota(jnp.int32, (L,), 0)
    for g in range(mpw // L):
        iv = cidx_v[pl.ds(g * L, L)]
        gx = plsc.load_gather(x_v, [iv])
        gy = plsc.load_gather(y_v, [iv])
        gz = plsc.load_gather(z_v, [iv])
        pos = iota + g * L
        plsc.store_scatter(nxb, [pos, jnp.zeros((L,), jnp.int32)], gx)
        plsc.store_scatter(nxb, [pos, jnp.ones((L,), jnp.int32)], gy)
        plsc.store_scatter(nxb, [pos, jnp.full((L,), 2, jnp.int32)], gz)
        sl = pl.ds(g * L, L)
        cxb_v[sl] = _round_bf16(gx)
        cyb_v[sl] = _round_bf16(gy)
        czb_v[sl] = _round_bf16(gz)
        cn_v[sl] = (gx * gx + gy * gy) + gz * gz
    pltpu.sync_copy(nxb, nxyz_out.at[pl.ds(b * M + mbase, mpw)])
    bm = B * M
    pltpu.sync_copy(cxb_v, aux_out.at[pl.ds(b * M + mbase, mpw)])
    pltpu.sync_copy(cyb_v, aux_out.at[pl.ds(bm + b * M + mbase, mpw)])
    pltpu.sync_copy(czb_v, aux_out.at[pl.ds(2 * bm + b * M + mbase, mpw)])
    pltpu.sync_copy(cn_v, aux_out.at[pl.ds(3 * bm + b * M + mbase, mpw)])


def _sc_select_body(B, N, M, mpw, xyzt, aux_hbm, g0, g1, rows0_out, rows1_out,
                    xb_v, yb_v, zb_v, pn_v, cxb_c, cyb_c, czb_c, cn_c,
                    ib0, ib1, idx0_v, idx1_v,
                    row0_a, row0_b, row1_a, row1_b,
                    gsem_a, gsem_b):
    wpb = NW // B
    wid = lax.axis_index("s") * NC + lax.axis_index("c")
    b = wid // wpb
    mbase = (wid % wpb) * mpw
    rbase = b * M + mbase
    bm = B * M

    pltpu.sync_copy(xyzt.at[pl.ds(b * 3 * N, N)], xb_v)
    pltpu.sync_copy(xyzt.at[pl.ds(b * 3 * N + N, N)], yb_v)
    pltpu.sync_copy(xyzt.at[pl.ds(b * 3 * N + 2 * N, N)], zb_v)
    pltpu.sync_copy(aux_hbm.at[pl.ds(rbase, mpw)], cxb_c)
    pltpu.sync_copy(aux_hbm.at[pl.ds(bm + rbase, mpw)], cyb_c)
    pltpu.sync_copy(aux_hbm.at[pl.ds(2 * bm + rbase, mpw)], czb_c)
    pltpu.sync_copy(aux_hbm.at[pl.ds(3 * bm + rbase, mpw)], cn_c)

    iota = lax.broadcasted_iota(jnp.int32, (L,), 0)
    nchunks = N // L
    nbase = b * N
    s0v = jnp.full((L,), S0, jnp.int32)
    s1v = jnp.full((L,), S1, jnp.int32)

    def prep(j, carry):
        sl = pl.ds(j * L, L)
        xs = xb_v[sl]
        ys = yb_v[sl]
        zs = zb_v[sl]
        pn_v[sl] = (xs * xs + ys * ys) + zs * zs
        xb_v[sl] = _round_bf16(xs)
        yb_v[sl] = _round_bf16(ys)
        zb_v[sl] = _round_bf16(zs)
        return carry

    lax.fori_loop(0, nchunks, prep, 0)

    def scan_center(i, cx, cy, cz, cn):
        def cond(c):
            j, w0, w1 = c
            return (j < nchunks // 2) & jnp.any((w0 < s0v) | (w1 < s1v))

        def one_chunk(jj, w0, w1):
            sl = pl.ds(jj * L, L)
            mm = (cx * xb_v[sl] + cy * yb_v[sl]) + cz * zb_v[sl]
            d2 = (cn + pn_v[sl]) - 2.0 * mm
            nv = iota + (jj * L)
            m0 = d2 < R0SQ
            m1 = d2 < R1SQ
            c0 = plsc.cumsum(m0.astype(jnp.int32))
            c1 = plsc.cumsum(m1.astype(jnp.int32))
            plsc.store_scatter(ib0, [jnp.minimum(w0 + c0 - 1, S0 + 31)],
                               nv, mask=m0)
            plsc.store_scatter(ib1, [jnp.minimum(w1 + c1 - 1, S1 + 31)],
                               nv, mask=m1)
            p0 = plsc.all_reduce_population_count(m0)
            p1 = plsc.all_reduce_population_count(m1)
            return w0 + p0, w1 + p1

        def bodyw(c):
            j, w0, w1 = c
            w0, w1 = one_chunk(2 * j, w0, w1)
            w0, w1 = one_chunk(2 * j + 1, w0, w1)
            return j + 1, w0, w1

        zero = jnp.zeros((L,), jnp.int32)
        _, w0f, w1f = lax.while_loop(cond, bodyw, (0, zero, zero))

        cnt0 = jnp.minimum(w0f, s0v)
        f0c = ib0[pl.ds(0, L)]
        first0 = jnp.sum(jnp.where(iota == 0, f0c, 0))
        for k in range(S0 // L):
            v = ib0[pl.ds(k * L, L)]
            posk = iota + k * L
            v = jnp.where(posk < cnt0, v, first0) + nbase
            idx0_v[pl.ds(i * S0 + k * L, L)] = v

        cnt1 = jnp.minimum(w1f, s1v)
        f1c = ib1[pl.ds(0, L)]
        first1 = jnp.sum(jnp.where(iota == 0, f1c, 0))
        for k in range(S1 // L):
            v = ib1[pl.ds(k * L, L)]
            posk = iota + k * L
            v = jnp.where(posk < cnt1, v, first1) + nbase
            idx1_v[pl.ds(i * S1 + k * L, L)] = v

    def center_body(i, carry):
        g = i // L
        lane = i % L
        sel = iota == lane
        gs = pl.ds(g * L, L)
        cx = jnp.sum(jnp.where(sel, cxb_c[gs], 0.0))
        cy = jnp.sum(jnp.where(sel, cyb_c[gs], 0.0))
        cz = jnp.sum(jnp.where(sel, czb_c[gs], 0.0))
        cn = jnp.sum(jnp.where(sel, cn_c[gs], 0.0))
        scan_center(i, cx, cy, cz, cn)
        return carry

    lax.fori_loop(0, mpw, center_body, 0)

    ch = 128

    def gat(idx_v, gtab, out, rowbase, rbuf_a, rbuf_b, nch):
        def gpair(k, carry):
            c0 = 2 * k
            c1 = 2 * k + 1
            ha = pltpu.async_copy(gtab.at[idx_v.at[pl.ds(c0 * ch, ch)]],
                                  rbuf_a, gsem_a)
            hb = pltpu.async_copy(gtab.at[idx_v.at[pl.ds(c1 * ch, ch)]],
                                  rbuf_b, gsem_b)
            ha.wait()
            pltpu.sync_copy(rbuf_a, out.at[pl.ds(rowbase + c0 * ch, ch)])
            hb.wait()
            pltpu.sync_copy(rbuf_b, out.at[pl.ds(rowbase + c1 * ch, ch)])
            return carry

        lax.fori_loop(0, nch // 2, gpair, 0)

    gat(idx0_v, g0, rows0_out, rbase * S0, row0_a, row0_b, mpw * S0 // ch)
    gat(idx1_v, g1, rows1_out, rbase * S1, row1_a, row1_b, mpw * S1 // ch)


def _fold_bn(layer):
    s = layer['gamma'] * lax.rsqrt(layer['var'] + BN_EPS)
    return layer['W'] * s[:, None], (layer['b'] - layer['mean']) * s + layer['beta']


def kernel(points_xyz, features, indices, params):
    B, N, _ = points_xyz.shape
    M = indices.shape[1]
    C = features.shape[1]

    w10, b10 = _fold_bn(params[0][0])
    w20, b20 = _fold_bn(params[0][1])
    w11, b11 = _fold_bn(params[1][0])
    w21, b21 = _fold_bn(params[1][1])
    w03 = jnp.transpose(w10[:, :3])
    wf0 = jnp.transpose(w10[:, 3:])
    w13 = jnp.transpose(w11[:, :3])
    wf1 = jnp.transpose(w11[:, 3:])
    w2t0 = jnp.transpose(w20)
    w2t1 = jnp.transpose(w21)

    xyzr = points_xyz.reshape(B * N, 3)
    featr = jnp.transpose(features, (0, 2, 1)).reshape(B * N, C)
    xyzt = jnp.transpose(points_xyz, (0, 2, 1)).reshape(B * 3 * N)

    mpw = (B * M) // NW
    mesh = plsc.VectorSubcoreMesh(core_axis_name="c", subcore_axis_name="s")

    sc_centers = pl.kernel(
        functools.partial(_sc_centers_body, B, N, M, mpw),
        out_type=(
            jax.ShapeDtypeStruct((B * M, 3), jnp.float32),
            jax.ShapeDtypeStruct((4 * B * M,), jnp.float32),
        ),
        mesh=mesh,
        compiler_params=pltpu.CompilerParams(needs_layout_passes=False),
        scratch_types=[
            pltpu.VMEM((N,), jnp.float32),
            pltpu.VMEM((N,), jnp.float32),
            pltpu.VMEM((N,), jnp.float32),
            pltpu.VMEM((mpw,), jnp.int32),
            pltpu.VMEM((mpw, 3), jnp.float32),
            pltpu.VMEM((mpw,), jnp.float32),
            pltpu.VMEM((mpw,), jnp.float32),
            pltpu.VMEM((mpw,), jnp.float32),
            pltpu.VMEM((mpw,), jnp.float32),
        ],
    )
    new_xyz_flat, aux = sc_centers(xyzt, indices.reshape(B * M))

    nb = 2048
    g0, g1 = pl.pallas_call(
        _pointfeat_body,
        grid=(B * N // nb,),
        in_specs=[
            pl.BlockSpec((nb, 3), lambda i: (i, 0)),
            pl.BlockSpec((nb, C), lambda i: (i, 0)),
            pl.BlockSpec((3, 64), lambda i: (0, 0)),
            pl.BlockSpec((C, 64), lambda i: (0, 0)),
            pl.BlockSpec((1, 64), lambda i: (0, 0)),
            pl.BlockSpec((3, 96), lambda i: (0, 0)),
            pl.BlockSpec((C, 96), lambda i: (0, 0)),
            pl.BlockSpec((1, 96), lambda i: (0, 0)),
        ],
        out_specs=[
            pl.BlockSpec((nb, 128), lambda i: (i, 0)),
            pl.BlockSpec((nb, 128), lambda i: (i, 0)),
        ],
        out_shape=[
            jax.ShapeDtypeStruct((B * N, 128), jnp.float32),
            jax.ShapeDtypeStruct((B * N, 128), jnp.float32),
        ],
    )(xyzr, featr, w03, wf0, b10.reshape(1, 64), w13, wf1, b11.reshape(1, 96))

    sc_select = pl.kernel(
        functools.partial(_sc_select_body, B, N, M, mpw),
        out_type=(
            jax.ShapeDtypeStruct((B * M * S0, 128), jnp.float32),
            jax.ShapeDtypeStruct((B * M * S1, 128), jnp.float32),
        ),
        mesh=mesh,
        compiler_params=pltpu.CompilerParams(needs_layout_passes=False),
        scratch_types=[
            pltpu.VMEM((N,), jnp.float32),
            pltpu.VMEM((N,), jnp.float32),
            pltpu.VMEM((N,), jnp.float32),
            pltpu.VMEM((N,), jnp.float32),
            pltpu.VMEM((mpw,), jnp.float32),
            pltpu.VMEM((mpw,), jnp.float32),
            pltpu.VMEM((mpw,), jnp.float32),
            pltpu.VMEM((mpw,), jnp.float32),
            pltpu.VMEM((S0 + 32,), jnp.int32),
            pltpu.VMEM((S1 + 32,), jnp.int32),
            pltpu.VMEM((mpw * S0,), jnp.int32),
            pltpu.VMEM((mpw * S1,), jnp.int32),
            pltpu.VMEM((128, 128), jnp.float32),
            pltpu.VMEM((128, 128), jnp.float32),
            pltpu.VMEM((128, 128), jnp.float32),
            pltpu.VMEM((128, 128), jnp.float32),
            pltpu.SemaphoreType.DMA,
            pltpu.SemaphoreType.DMA,
        ],
    )
    rows0, rows1 = sc_select(xyzt, aux, g0, g1)

    mb = 128
    m_blocks = (B * M) // mb
    out = pl.pallas_call(
        functools.partial(_head_body, mb),
        grid=(m_blocks,),
        in_specs=[
            pl.BlockSpec((mb, 3), lambda i: (i, 0)),
            pl.BlockSpec((mb * S0, 128), lambda i: (i, 0)),
            pl.BlockSpec((mb * S1, 128), lambda i: (i, 0)),
            pl.BlockSpec((3, 64), lambda i: (0, 0)),
            pl.BlockSpec((3, 96), lambda i: (0, 0)),
            pl.BlockSpec((64, 128), lambda i: (0, 0)),
            pl.BlockSpec((1, 128), lambda i: (0, 0)),
            pl.BlockSpec((96, 128), lambda i: (0, 0)),
            pl.BlockSpec((1, 128), lambda i: (0, 0)),
        ],
        out_specs=pl.BlockSpec(
            (1, 256, mb),
            lambda i, _mblk=M // mb: (i // _mblk, 0, i % _mblk)),
        out_shape=jax.ShapeDtypeStruct((B, 256, M), jnp.float32),
    )(new_xyz_flat, rows0, rows1, w03, w13,
      w2t0, b20.reshape(1, 128), w2t1, b21.reshape(1, 128))

    return (new_xyz_flat.reshape(B, M, 3), out, indices)


# speedup vs baseline: 50.3404x; 1.0417x over previous
import functools

import jax
import jax.numpy as jnp
from jax import lax
from jax.experimental import pallas as pl
from jax.experimental.pallas import tpu as pltpu
from jax.experimental.pallas import tpu_sc as plsc

R0SQ = 0.2 * 0.2
R1SQ = 0.4 * 0.4
S0 = 32
S1 = 64
BN_EPS = 1e-5
NC = 2
NSUB = 16
NW = NC * NSUB
L = 16


def _pointfeat_body(x_ref, f_ref, w03_ref, wf0_ref, b0_ref, w13_ref, wf1_ref,
                    b1_ref, g0_ref, g1_ref):
    x = x_ref[...]
    f = f_ref[...]
    g0 = jnp.dot(x, w03_ref[...], preferred_element_type=jnp.float32)
    g0 = g0 + jnp.dot(f, wf0_ref[...], preferred_element_type=jnp.float32)
    g0_ref[:, :64] = g0 + b0_ref[...]
    g1 = jnp.dot(x, w13_ref[...], preferred_element_type=jnp.float32)
    g1 = g1 + jnp.dot(f, wf1_ref[...], preferred_element_type=jnp.float32)
    g1_ref[:, :96] = g1 + b1_ref[...]


def _head_body(mb, nx_ref, r0_ref, r1_ref, w03_ref, w13_ref,
               w20_ref, b20_ref, w21_ref, b21_ref, o_ref):
    nx = nx_ref[...]
    t0 = jnp.dot(nx, w03_ref[...], preferred_element_type=jnp.float32)
    r0 = r0_ref[:, :64].reshape(mb, S0, 64)
    a0 = jnp.maximum(r0 - t0[:, None, :], 0.0).reshape(mb * S0, 64)
    h0 = jnp.dot(a0, w20_ref[...], preferred_element_type=jnp.float32)
    h0 = jnp.maximum(h0 + b20_ref[...], 0.0).reshape(mb, S0, 128)
    y0 = jnp.max(h0, axis=1)

    t1 = jnp.dot(nx, w13_ref[...], preferred_element_type=jnp.float32)
    r1 = r1_ref[:, :96].reshape(mb, S1, 96)
    a1 = jnp.maximum(r1 - t1[:, None, :], 0.0).reshape(mb * S1, 96)
    h1 = jnp.dot(a1, w21_ref[...], preferred_element_type=jnp.float32)
    h1 = jnp.maximum(h1 + b21_ref[...], 0.0).reshape(mb, S1, 128)
    y1 = jnp.max(h1, axis=1)

    y = jnp.concatenate([y0, y1], axis=1)
    o_ref[...] = y.T[None]


def _round_bf16(x):
    u = plsc.bitcast(x, jnp.int32)
    r = (u + 0x7FFF + (lax.shift_right_logical(u, 16) & 1)) & (-65536)
    return plsc.bitcast(r, jnp.float32)


def _sc_centers_body(B, N, M, mpw, xyzt, cidx_hbm, nxyz_out, aux_out,
                     x_v, y_v, z_v, cidx_v, nxb, cxb_v, cyb_v, czb_v, cn_v):
    wpb = NW // B
    wid = lax.axis_index("s") * NC + lax.axis_index("c")
    b = wid // wpb
    mbase = (wid % wpb) * mpw

    pltpu.sync_copy(xyzt.at[pl.ds(b * 3 * N, N)], x_v)
    pltpu.sync_copy(xyzt.at[pl.ds(b * 3 * N + N, N)], y_v)
    pltpu.sync_copy(xyzt.at[pl.ds(b * 3 * N + 2 * N, N)], z_v)
    pltpu.sync_copy(cidx_hbm.at[pl.ds(b * M + mbase, mpw)], cidx_v)

    iota = lax.broadcasted_iota(jnp.int32, (L,), 0)
    for g in range(mpw // L):
        iv = cidx_v[pl.ds(g * L, L)]
        gx = plsc.load_gather(x_v, [iv])
        gy = plsc.load_gather(y_v, [iv])
        gz = plsc.load_gather(z_v, [iv])
        pos = iota + g * L
        plsc.store_scatter(nxb, [pos, jnp.zeros((L,), jnp.int32)], gx)
        plsc.store_scatter(nxb, [pos, jnp.ones((L,), jnp.int32)], gy)
        plsc.store_scatter(nxb, [pos, jnp.full((L,), 2, jnp.int32)], gz)
        sl = pl.ds(g * L, L)
        cxb_v[sl] = _round_bf16(gx)
        cyb_v[sl] = _round_bf16(gy)
        czb_v[sl] = _round_bf16(gz)
        cn_v[sl] = (gx * gx + gy * gy) + gz * gz
    pltpu.sync_copy(nxb, nxyz_out.at[pl.ds(b * M + mbase, mpw)])
    bm = B * M
    pltpu.sync_copy(cxb_v, aux_out.at[pl.ds(b * M + mbase, mpw)])
    pltpu.sync_copy(cyb_v, aux_out.at[pl.ds(bm + b * M + mbase, mpw)])
    pltpu.sync_copy(czb_v, aux_out.at[pl.ds(2 * bm + b * M + mbase, mpw)])
    pltpu.sync_copy(cn_v, aux_out.at[pl.ds(3 * bm + b * M + mbase, mpw)])


def _sc_select_body(B, N, M, mpw, xyzt, aux_hbm, g0, g1, rows0_out, rows1_out,
                    xb_v, yb_v, zb_v, pn_v, cxb_c, cyb_c, czb_c, cn_c,
                    ib0, ib1, idx0_v, idx1_v,
                    row0_a, row0_b, row1_a, row1_b,
                    gsem_a, gsem_b):
    wpb = NW // B
    wid = lax.axis_index("s") * NC + lax.axis_index("c")
    b = wid // wpb
    mbase = (wid % wpb) * mpw
    rbase = b * M + mbase
    bm = B * M

    pltpu.sync_copy(xyzt.at[pl.ds(b * 3 * N, N)], xb_v)
    pltpu.sync_copy(xyzt.at[pl.ds(b * 3 * N + N, N)], yb_v)
    pltpu.sync_copy(xyzt.at[pl.ds(b * 3 * N + 2 * N, N)], zb_v)
    pltpu.sync_copy(aux_hbm.at[pl.ds(rbase, mpw)], cxb_c)
    pltpu.sync_copy(aux_hbm.at[pl.ds(bm + rbase, mpw)], cyb_c)
    pltpu.sync_copy(aux_hbm.at[pl.ds(2 * bm + rbase, mpw)], czb_c)
    pltpu.sync_copy(aux_hbm.at[pl.ds(3 * bm + rbase, mpw)], cn_c)

    iota = lax.broadcasted_iota(jnp.int32, (L,), 0)
    nchunks = N // L
    nbase = b * N
    s0v = jnp.full((L,), S0, jnp.int32)
    s1v = jnp.full((L,), S1, jnp.int32)

    def prep(j, carry):
        sl = pl.ds(j * L, L)
        xs = xb_v[sl]
        ys = yb_v[sl]
        zs = zb_v[sl]
        pn_v[sl] = (xs * xs + ys * ys) + zs * zs
        xb_v[sl] = _round_bf16(xs)
        yb_v[sl] = _round_bf16(ys)
        zb_v[sl] = _round_bf16(zs)
        return carry

    lax.fori_loop(0, nchunks, prep, 0)

    def scan_center(i, cx, cy, cz, cn):
        def cond(c):
            j, w0, w1 = c
            return (j < nchunks // 4) & jnp.any((w0 < s0v) | (w1 < s1v))

        def one_chunk(jj, w0, w1):
            sl = pl.ds(jj * L, L)
            mm = (cx * xb_v[sl] + cy * yb_v[sl]) + cz * zb_v[sl]
            d2 = (cn + pn_v[sl]) - 2.0 * mm
            nv = iota + (jj * L)
            m0 = d2 < R0SQ
            m1 = d2 < R1SQ
            c0 = plsc.cumsum(m0.astype(jnp.int32))
            c1 = plsc.cumsum(m1.astype(jnp.int32))
            plsc.store_scatter(ib0, [jnp.minimum(w0 + c0 - 1, S0 + 31)],
                               nv, mask=m0)
            plsc.store_scatter(ib1, [jnp.minimum(w1 + c1 - 1, S1 + 31)],
                               nv, mask=m1)
            p0 = plsc.all_reduce_population_count(m0)
            p1 = plsc.all_reduce_population_count(m1)
            return w0 + p0, w1 + p1

        def bodyw(c):
            j, w0, w1 = c
            w0, w1 = one_chunk(4 * j, w0, w1)
            w0, w1 = one_chunk(4 * j + 1, w0, w1)
            w0, w1 = one_chunk(4 * j + 2, w0, w1)
            w0, w1 = one_chunk(4 * j + 3, w0, w1)
            return j + 1, w0, w1

        zero = jnp.zeros((L,), jnp.int32)
        _, w0f, w1f = lax.while_loop(cond, bodyw, (0, zero, zero))

        cnt0 = jnp.minimum(w0f, s0v)
        f0c = ib0[pl.ds(0, L)]
        first0 = jnp.sum(jnp.where(iota == 0, f0c, 0))
        for k in range(S0 // L):
            v = ib0[pl.ds(k * L, L)]
            posk = iota + k * L
            v = jnp.where(posk < cnt0, v, first0) + nbase
            idx0_v[pl.ds(i * S0 + k * L, L)] = v

        cnt1 = jnp.minimum(w1f, s1v)
        f1c = ib1[pl.ds(0, L)]
        first1 = jnp.sum(jnp.where(iota == 0, f1c, 0))
        for k in range(S1 // L):
            v = ib1[pl.ds(k * L, L)]
            posk = iota + k * L
            v = jnp.where(posk < cnt1, v, first1) + nbase
            idx1_v[pl.ds(i * S1 + k * L, L)] = v

    def center_body(i, carry):
        g = i // L
        lane = i % L
        sel = iota == lane
        gs = pl.ds(g * L, L)
        cx = jnp.sum(jnp.where(sel, cxb_c[gs], 0.0))
        cy = jnp.sum(jnp.where(sel, cyb_c[gs], 0.0))
        cz = jnp.sum(jnp.where(sel, czb_c[gs], 0.0))
        cn = jnp.sum(jnp.where(sel, cn_c[gs], 0.0))
        scan_center(i, cx, cy, cz, cn)
        return carry

    lax.fori_loop(0, mpw, center_body, 0)

    ch = 128

    def gat(idx_v, gtab, out, rowbase, rbuf_a, rbuf_b, nch):
        def gpair(k, carry):
            c0 = 2 * k
            c1 = 2 * k + 1
            ha = pltpu.async_copy(gtab.at[idx_v.at[pl.ds(c0 * ch, ch)]],
                                  rbuf_a, gsem_a)
            hb = pltpu.async_copy(gtab.at[idx_v.at[pl.ds(c1 * ch, ch)]],
                                  rbuf_b, gsem_b)
            ha.wait()
            pltpu.sync_copy(rbuf_a, out.at[pl.ds(rowbase + c0 * ch, ch)])
            hb.wait()
            pltpu.sync_copy(rbuf_b, out.at[pl.ds(rowbase + c1 * ch, ch)])
            return carry

        lax.fori_loop(0, nch // 2, gpair, 0)

    gat(idx0_v, g0, rows0_out, rbase * S0, row0_a, row0_b, mpw * S0 // ch)
    gat(idx1_v, g1, rows1_out, rbase * S1, row1_a, row1_b, mpw * S1 // ch)


def _fold_bn(layer):
    s = layer['gamma'] * lax.rsqrt(layer['var'] + BN_EPS)
    return layer['W'] * s[:, None], (layer['b'] - layer['mean']) * s + layer['beta']


def kernel(points_xyz, features, indices, params):
    B, N, _ = points_xyz.shape
    M = indices.shape[1]
    C = features.shape[1]

    w10, b10 = _fold_bn(params[0][0])
    w20, b20 = _fold_bn(params[0][1])
    w11, b11 = _fold_bn(params[1][0])
    w21, b21 = _fold_bn(params[1][1])
    w03 = jnp.transpose(w10[:, :3])
    wf0 = jnp.transpose(w10[:, 3:])
    w13 = jnp.transpose(w11[:, :3])
    wf1 = jnp.transpose(w11[:, 3:])
    w2t0 = jnp.transpose(w20)
    w2t1 = jnp.transpose(w21)

    xyzr = points_xyz.reshape(B * N, 3)
    featr = jnp.transpose(features, (0, 2, 1)).reshape(B * N, C)
    xyzt = jnp.transpose(points_xyz, (0, 2, 1)).reshape(B * 3 * N)

    mpw = (B * M) // NW
    mesh = plsc.VectorSubcoreMesh(core_axis_name="c", subcore_axis_name="s")

    sc_centers = pl.kernel(
        functools.partial(_sc_centers_body, B, N, M, mpw),
        out_type=(
            jax.ShapeDtypeStruct((B * M, 3), jnp.float32),
            jax.ShapeDtypeStruct((4 * B * M,), jnp.float32),
        ),
        mesh=mesh,
        compiler_params=pltpu.CompilerParams(needs_layout_passes=False),
        scratch_types=[
            pltpu.VMEM((N,), jnp.float32),
            pltpu.VMEM((N,), jnp.float32),
            pltpu.VMEM((N,), jnp.float32),
            pltpu.VMEM((mpw,), jnp.int32),
            pltpu.VMEM((mpw, 3), jnp.float32),
            pltpu.VMEM((mpw,), jnp.float32),
            pltpu.VMEM((mpw,), jnp.float32),
            pltpu.VMEM((mpw,), jnp.float32),
            pltpu.VMEM((mpw,), jnp.float32),
        ],
    )
    new_xyz_flat, aux = sc_centers(xyzt, indices.reshape(B * M))

    nb = 2048
    g0, g1 = pl.pallas_call(
        _pointfeat_body,
        grid=(B * N // nb,),
        in_specs=[
            pl.BlockSpec((nb, 3), lambda i: (i, 0)),
            pl.BlockSpec((nb, C), lambda i: (i, 0)),
            pl.BlockSpec((3, 64), lambda i: (0, 0)),
            pl.BlockSpec((C, 64), lambda i: (0, 0)),
            pl.BlockSpec((1, 64), lambda i: (0, 0)),
            pl.BlockSpec((3, 96), lambda i: (0, 0)),
            pl.BlockSpec((C, 96), lambda i: (0, 0)),
            pl.BlockSpec((1, 96), lambda i: (0, 0)),
        ],
        out_specs=[
            pl.BlockSpec((nb, 128), lambda i: (i, 0)),
            pl.BlockSpec((nb, 128), lambda i: (i, 0)),
        ],
        out_shape=[
            jax.ShapeDtypeStruct((B * N, 128), jnp.float32),
            jax.ShapeDtypeStruct((B * N, 128), jnp.float32),
        ],
    )(xyzr, featr, w03, wf0, b10.reshape(1, 64), w13, wf1, b11.reshape(1, 96))

    sc_select = pl.kernel(
        functools.partial(_sc_select_body, B, N, M, mpw),
        out_type=(
            jax.ShapeDtypeStruct((B * M * S0, 128), jnp.float32),
            jax.ShapeDtypeStruct((B * M * S1, 128), jnp.float32),
        ),
        mesh=mesh,
        compiler_params=pltpu.CompilerParams(needs_layout_passes=False),
        scratch_types=[
            pltpu.VMEM((N,), jnp.float32),
            pltpu.VMEM((N,), jnp.float32),
            pltpu.VMEM((N,), jnp.float32),
            pltpu.VMEM((N,), jnp.float32),
            pltpu.VMEM((mpw,), jnp.float32),
            pltpu.VMEM((mpw,), jnp.float32),
            pltpu.VMEM((mpw,), jnp.float32),
            pltpu.VMEM((mpw,), jnp.float32),
            pltpu.VMEM((S0 + 32,), jnp.int32),
            pltpu.VMEM((S1 + 32,), jnp.int32),
            pltpu.VMEM((mpw * S0,), jnp.int32),
            pltpu.VMEM((mpw * S1,), jnp.int32),
            pltpu.VMEM((128, 128), jnp.float32),
            pltpu.VMEM((128, 128), jnp.float32),
            pltpu.VMEM((128, 128), jnp.float32),
            pltpu.VMEM((128, 128), jnp.float32),
            pltpu.SemaphoreType.DMA,
            pltpu.SemaphoreType.DMA,
        ],
    )
    rows0, rows1 = sc_select(xyzt, aux, g0, g1)

    mb = 128
    m_blocks = (B * M) // mb
    out = pl.pallas_call(
        functools.partial(_head_body, mb),
        grid=(m_blocks,),
        in_specs=[
            pl.BlockSpec((mb, 3), lambda i: (i, 0)),
            pl.BlockSpec((mb * S0, 128), lambda i: (i, 0)),
            pl.BlockSpec((mb * S1, 128), lambda i: (i, 0)),
            pl.BlockSpec((3, 64), lambda i: (0, 0)),
            pl.BlockSpec((3, 96), lambda i: (0, 0)),
            pl.BlockSpec((64, 128), lambda i: (0, 0)),
            pl.BlockSpec((1, 128), lambda i: (0, 0)),
            pl.BlockSpec((96, 128), lambda i: (0, 0)),
            pl.BlockSpec((1, 128), lambda i: (0, 0)),
        ],
        out_specs=pl.BlockSpec(
            (1, 256, mb),
            lambda i, _mblk=M // mb: (i // _mblk, 0, i % _mblk)),
        out_shape=jax.ShapeDtypeStruct((B, 256, M), jnp.float32),
    )(new_xyz_flat, rows0, rows1, w03, w13,
      w2t0, b20.reshape(1, 128), w2t1, b21.reshape(1, 128))

    return (new_xyz_flat.reshape(B, M, 3), out, indices)
